# baseline (speedup 1.0000x reference)
import functools

import jax
import jax.numpy as jnp
from jax import lax
from jax.experimental import pallas as pl
from jax.experimental.pallas import tpu as pltpu
from jax.experimental.pallas import tpu_sc as plsc

VOCAB = 100000
EMBED = 64
HIDDEN = 128
NUM_CLASSES = 2
B = 4096
L = 200

NC = 2
NS = 16
NW = NC * NS
BPW = B // NW
C1 = 128
C2 = L - C1
NVEC = EMBED // 16


def _pool_body(x_hbm, emb_hbm, out_hbm, idx_v, rows_v, pooled_v, sem0, sem1):
    wid = lax.axis_index("s") * NC + lax.axis_index("c")
    base = wid * BPW

    pltpu.sync_copy(x_hbm.at[pl.ds(base, BPW)], idx_v)

    def copies(s, b):
        rbuf = rows_v.at[b]
        sem = sem0 if b == 0 else sem1
        h1 = pltpu.make_async_copy(
            emb_hbm.at[idx_v.at[s, pl.ds(0, C1)]], rbuf.at[pl.ds(0, C1)], sem)
        h2 = pltpu.make_async_copy(
            emb_hbm.at[idx_v.at[s, pl.ds(C1, C2)]], rbuf.at[pl.ds(C1, C2)], sem)
        return h1, h2

    def fire(s, b):
        h1, h2 = copies(s, b)
        h1.start()
        h2.start()

    def drain(s, b):
        h1, h2 = copies(s, b)
        h1.wait()
        h2.wait()

    def accum(s, b):
        rbuf = rows_v.at[b]

        def body(j, accs):
            return tuple(accs[i] + rbuf[j, pl.ds(16 * i, 16)]
                         for i in range(NVEC))

        accs = lax.fori_loop(
            0, L, body,
            tuple(jnp.zeros((16,), jnp.float32) for _ in range(NVEC)),
            unroll=8)
        inv = jnp.float32(1.0 / L)
        for i in range(NVEC):
            pooled_v[s, pl.ds(16 * i, 16)] = accs[i] * inv

    fire(0, 0)
    fire(1, 1)

    def outer(g, carry):
        for b in range(2):
            s = 2 * g + b
            drain(s, b)

            @pl.when(s + 2 < BPW)
            def _():
                fire(s + 2, b)

            accum(s, b)
        return carry

    lax.fori_loop(0, BPW // 2, outer, 0)

    pltpu.sync_copy(pooled_v, out_hbm.at[pl.ds(base, BPW)])


_pool = pl.kernel(
    _pool_body,
    out_type=jax.ShapeDtypeStruct((B, EMBED), jnp.float32),
    mesh=plsc.VectorSubcoreMesh(core_axis_name="c", subcore_axis_name="s"),
    scratch_types=[
        pltpu.VMEM((BPW, L), jnp.int32),
        pltpu.VMEM((2, L, EMBED), jnp.float32),
        pltpu.VMEM((BPW, EMBED), jnp.float32),
        pltpu.SemaphoreType.DMA,
        pltpu.SemaphoreType.DMA,
    ],
    compiler_params=pltpu.CompilerParams(use_tc_tiling_on_sc=False),
)


def _mlp_body(p_ref, w1_ref, b1_ref, w2_ref, b2_ref, o_ref):
    p = p_ref[:]
    h = lax.dot_general(p, w1_ref[:], (((1,), (1,)), ((), ())),
                        preferred_element_type=jnp.float32)
    h = jnp.maximum(h + b1_ref[:], 0.0)
    o = lax.dot_general(h, w2_ref[:], (((1,), (1,)), ((), ())),
                        preferred_element_type=jnp.float32)
    o_ref[:] = o + b2_ref[:]


_mlp = pl.pallas_call(
    _mlp_body,
    out_shape=jax.ShapeDtypeStruct((B, NUM_CLASSES), jnp.float32),
)


def kernel(x, emb, W1, b1, W2, b2):
    x = x.astype(jnp.int32)
    pooled = _pool(x, emb)
    return pooled[:, :NUM_CLASSES] * 1.0


# speedup vs baseline: 1.0354x; 1.0354x over previous
import functools

import jax
import jax.numpy as jnp
from jax import lax
from jax.experimental import pallas as pl
from jax.experimental.pallas import tpu as pltpu
from jax.experimental.pallas import tpu_sc as plsc

VOCAB = 100000
EMBED = 64
HIDDEN = 128
NUM_CLASSES = 2
B = 4096
L = 200

NC = 2
NS = 16
NW = NC * NS
BPW = B // NW
C1 = 128
C2 = L - C1
NVEC = EMBED // 16
NBUF = 4


def _pool_body(x_hbm, emb_hbm, out_hbm, idx_v, rows_v, pooled_v, *sems):
    wid = lax.axis_index("s") * NC + lax.axis_index("c")
    base = wid * BPW

    pltpu.sync_copy(x_hbm.at[pl.ds(base, BPW)], idx_v)

    def copies(s, b):
        rbuf = rows_v.at[b]
        sem = sems[b]
        h1 = pltpu.make_async_copy(
            emb_hbm.at[idx_v.at[s, pl.ds(0, C1)]], rbuf.at[pl.ds(0, C1)], sem)
        h2 = pltpu.make_async_copy(
            emb_hbm.at[idx_v.at[s, pl.ds(C1, C2)]], rbuf.at[pl.ds(C1, C2)], sem)
        return h1, h2

    def fire(s, b):
        h1, h2 = copies(s, b)
        h1.start()
        h2.start()

    def drain(s, b):
        h1, h2 = copies(s, b)
        h1.wait()
        h2.wait()

    def accum(s, b):
        rbuf = rows_v.at[b]

        def body(j, accs):
            return tuple(accs[i] + rbuf[j, pl.ds(16 * i, 16)]
                         for i in range(NVEC))

        accs = lax.fori_loop(
            0, L, body,
            tuple(jnp.zeros((16,), jnp.float32) for _ in range(NVEC)),
            unroll=8)
        inv = jnp.float32(1.0 / L)
        for i in range(NVEC):
            pooled_v[s, pl.ds(16 * i, 16)] = accs[i] * inv

    for b in range(NBUF):
        fire(b, b)

    def outer(g, carry):
        for b in range(NBUF):
            s = NBUF * g + b
            drain(s, b)

            @pl.when(s + NBUF < BPW)
            def _():
                fire(s + NBUF, b)

            accum(s, b)
        return carry

    lax.fori_loop(0, BPW // NBUF, outer, 0)

    pltpu.sync_copy(pooled_v, out_hbm.at[pl.ds(base, BPW)])


_pool = pl.kernel(
    _pool_body,
    out_type=jax.ShapeDtypeStruct((B, EMBED), jnp.float32),
    mesh=plsc.VectorSubcoreMesh(core_axis_name="c", subcore_axis_name="s",
                                num_cores=NC, num_subcores=NS),
    scratch_types=[
        pltpu.VMEM((BPW, L), jnp.int32),
        pltpu.VMEM((NBUF, L, EMBED), jnp.float32),
        pltpu.VMEM((BPW, EMBED), jnp.float32),
    ] + [pltpu.SemaphoreType.DMA] * NBUF,
    compiler_params=pltpu.CompilerParams(use_tc_tiling_on_sc=False),
)


def _mlp_body(p_ref, w1_ref, b1_ref, w2_ref, b2_ref, o_ref):
    p = p_ref[:]
    h = lax.dot_general(p, w1_ref[:], (((1,), (1,)), ((), ())),
                        preferred_element_type=jnp.float32)
    h = jnp.maximum(h + b1_ref[:], 0.0)
    o = lax.dot_general(h, w2_ref[:], (((1,), (1,)), ((), ())),
                        preferred_element_type=jnp.float32)
    o_ref[:] = o + b2_ref[:]


_mlp = pl.pallas_call(
    _mlp_body,
    out_shape=jax.ShapeDtypeStruct((B, NUM_CLASSES), jnp.float32),
)


def kernel(x, emb, W1, b1, W2, b2):
    x = x.astype(jnp.int32)
    pooled = _pool(x, emb)
    return _mlp(pooled, W1, b1.reshape(1, HIDDEN), W2, b2.reshape(1, NUM_CLASSES))
